# Initial kernel scaffold; baseline (speedup 1.0000x reference)
import jax
import jax.numpy as jnp
from jax.experimental import pallas as pl


def kernel(difference, weight, epoch, iteration):
    raise NotImplementedError("write your pallas kernel here")


# speedup vs baseline: 6.3697x; 6.3697x over previous
import functools

import jax
import jax.numpy as jnp
from jax import lax
from jax.experimental import pallas as pl
from jax.experimental.pallas import tpu as pltpu
from jax.experimental.pallas import tpu_sc as plsc

B, N = 64, 8192
TOP_N = 500
L = 16
NV = N // L
NBKT = 2048
NBG = NBKT // L
ROWS_PER_W = 2

_mesh = plsc.VectorSubcoreMesh(core_axis_name="c", subcore_axis_name="s")


@functools.partial(
    pl.kernel,
    mesh=_mesh,
    out_type=jax.ShapeDtypeStruct((B, N), jnp.float32),
    compiler_params=pltpu.CompilerParams(needs_layout_passes=False),
    scratch_types=[
        pltpu.VMEM((N,), jnp.float32),
        pltpu.VMEM((N,), jnp.int32),
        pltpu.VMEM((N,), jnp.float32),
        pltpu.VMEM((N,), jnp.float32),
        pltpu.VMEM((NBKT,), jnp.int32),
        pltpu.VMEM((N,), jnp.int32),
        pltpu.VMEM((L,), jnp.float32),
    ],
)
def _sc_topk_mask(diff_hbm, cond_hbm, weight_hbm, out_hbm,
                  d_ref, bits_ref, w_ref, o_ref, hist_ref, cand_ref, cond_ref):
    wid = lax.axis_index("c") * 16 + lax.axis_index("s")
    pltpu.sync_copy(cond_hbm, cond_ref)
    condv = cond_ref[...]
    iota = lax.iota(jnp.int32, L)
    ones = jnp.ones((L,), jnp.int32)

    for r in range(ROWS_PER_W):
        row = wid * ROWS_PER_W + r
        pltpu.sync_copy(diff_hbm.at[row], d_ref)
        pltpu.sync_copy(weight_hbm.at[row], w_ref)

        def _zero(j, _):
            hist_ref[pl.ds(j * L, L)] = jnp.zeros((L,), jnp.int32)
            return 0
        lax.fori_loop(0, NBG, _zero, 0)

        def _hist(i, _):
            v = d_ref[pl.ds(i * L, L)]
            b = lax.bitcast_convert_type(v, jnp.int32) & 0x7FFFFFFF
            bits_ref[pl.ds(i * L, L)] = b
            plsc.addupdate_scatter(hist_ref, [b >> 20], ones)
            return 0
        lax.fori_loop(0, NV, _hist, 0)

        def _scan(j, carry):
            cum, E, needE = carry
            h = hist_ref[pl.ds((NBG - 1 - j) * L, L)]
            hd = lax.rev(h, (0,))
            inc = jnp.cumsum(hd)
            tot = cum + inc
            crossed = tot >= TOP_N
            lane = jnp.min(jnp.where(crossed, iota, L))
            found = (E < 0) & (lane < L)
            inc_l = jnp.sum(jnp.where(iota == lane, inc, 0))
            hd_l = jnp.sum(jnp.where(iota == lane, hd, 0))
            E = jnp.where(found, NBKT - 1 - j * L - lane, E)
            needE = jnp.where(found, TOP_N - (cum + inc_l - hd_l), needE)
            return cum + jnp.sum(h), E, needE
        _, E, needE = lax.fori_loop(
            0, NBG, _scan, (jnp.int32(0), jnp.int32(-1), jnp.int32(0)))

        def _compact(i, off):
            b = bits_ref[pl.ds(i * L, L)]
            m = (b >> 20) == E
            mi = m.astype(jnp.int32)
            exc = jnp.cumsum(mi) - mi
            plsc.store_scatter(cand_ref, [off + exc], b, mask=m)
            return off + jnp.sum(mi)
        K = lax.fori_loop(0, NV, _compact, jnp.int32(0))
        nv_c = (K + (L - 1)) // L

        prefix = E << 20

        def _count_ge(T):
            def _cnt(i, c):
                v = cand_ref[pl.ds(i * L, L)]
                valid = (i * L + iota) < K
                return c + jnp.sum(jnp.where((v >= T) & valid, 1, 0))
            return lax.fori_loop(0, nv_c, _cnt, jnp.int32(0))

        def _bs(s, lo):
            bit = lax.shift_left(jnp.int32(1), 19 - s)
            T = prefix | lo | bit
            c = _count_ge(T)
            return jnp.where(c >= needE, lo | bit, lo)
        lo = lax.fori_loop(0, 20, _bs, jnp.int32(0))
        t = prefix | lo
        need_eq = needE - _count_ge(t + 1)

        def _out(i, run):
            b = bits_ref[pl.ds(i * L, L)]
            wv = w_ref[pl.ds(i * L, L)]
            gt = b > t
            eq = b == t
            eqi = eq.astype(jnp.int32)
            exc = jnp.cumsum(eqi) - eqi
            sel = gt | (eq & ((run + exc) < need_eq))
            o_ref[pl.ds(i * L, L)] = wv + jnp.where(sel, condv, 0.0)
            return run + jnp.sum(eqi)
        lax.fori_loop(0, NV, _out, jnp.int32(0))

        pltpu.sync_copy(o_ref, out_hbm.at[row])


def kernel(difference, weight, epoch, iteration):
    cond = (epoch > 1) & (epoch % 2 == 0)
    condf = jnp.where(cond, jnp.float32(1.0), jnp.float32(0.0))
    cond16 = jnp.broadcast_to(condf, (L,))
    return _sc_topk_mask(difference, cond16, weight)
